# baseline (speedup 1.0000x reference)
import jax
import jax.numpy as jnp
from jax.experimental import pallas as pl
from jax.experimental.pallas import tpu as pltpu

_NUM_EXPERTS = 8
_TOP_K = 2
_EMBED = 768
_N = 32768
_BLK = 4096


def _router_block(x_ref, w_ref, out_ref, psum_acc, cnt_acc):
    i = pl.program_id(0)

    @pl.when(i == 0)
    def _init():
        psum_acc[...] = jnp.zeros_like(psum_acc)
        cnt_acc[...] = jnp.zeros_like(cnt_acc)

    x = x_ref[...]
    w = w_ref[...]
    logits = jax.lax.dot_general(
        w, x, (((1,), (1,)), ((), ())), preferred_element_type=jnp.float32
    )

    ids = jax.lax.broadcasted_iota(jnp.int32, logits.shape, 0).astype(jnp.float32)
    m1 = jnp.max(logits, axis=0, keepdims=True)
    a1 = jnp.min(jnp.where(logits == m1, ids, 8.0), axis=0,
                 keepdims=True)
    masked = jnp.where(ids == a1, -jnp.inf, logits)
    m2 = jnp.max(masked, axis=0, keepdims=True)
    a2 = jnp.min(jnp.where(masked == m2, ids, 8.0), axis=0,
                 keepdims=True)

    g = jnp.exp(m2 - m1)
    rden = 1.0 / (1.0 + g)
    w1 = rden
    w2 = g * rden

    p = jnp.exp(logits - m1)
    p = p * (1.0 / jnp.sum(p, axis=0, keepdims=True))
    psum_acc[...] += jnp.sum(p, axis=1, keepdims=True)
    onehot = (ids == a1).astype(jnp.float32) + (ids == a2).astype(jnp.float32)
    cnt_acc[...] += jnp.sum(onehot, axis=1, keepdims=True)

    f = cnt_acc[...] / (_N * _TOP_K)
    pmean = psum_acc[...] / _N
    aux_row = jnp.full((1, _BLK), _NUM_EXPERTS * jnp.sum(f * pmean),
                       jnp.float32)
    out_ref[...] = jnp.concatenate(
        [w1, w2, a1, a2, aux_row, aux_row, aux_row, aux_row], axis=0)


@jax.jit
def kernel(hidden_states, W):
    grid = (_N // _BLK,)
    packed = pl.pallas_call(
        _router_block,
        grid=grid,
        in_specs=[
            pl.BlockSpec((_BLK, _EMBED), lambda i: (i, 0)),
            pl.BlockSpec((_NUM_EXPERTS, _EMBED), lambda i: (0, 0)),
        ],
        out_specs=pl.BlockSpec((8, _BLK), lambda i: (0, i)),
        out_shape=jax.ShapeDtypeStruct((8, _N), jnp.float32),
        scratch_shapes=[
            pltpu.VMEM((_NUM_EXPERTS, 1), jnp.float32),
            pltpu.VMEM((_NUM_EXPERTS, 1), jnp.float32),
        ],
    )(hidden_states, W)
    wts = packed[0:2].T
    exps = packed[2:4].T.astype(jnp.int32)
    return wts, exps, packed[4, _N - 1]


# speedup vs baseline: 1.0437x; 1.0437x over previous
import jax
import jax.numpy as jnp
from jax.experimental import pallas as pl
from jax.experimental.pallas import tpu as pltpu

_NUM_EXPERTS = 8
_TOP_K = 2
_EMBED = 768
_N = 32768
_BLK = 4096


def _router_block(x_ref, w_ref, out_ref, aux_ref, psum_acc, cnt_acc):
    i = pl.program_id(0)
    nsteps = pl.num_programs(0)

    @pl.when(i == 0)
    def _init():
        psum_acc[...] = jnp.zeros_like(psum_acc)
        cnt_acc[...] = jnp.zeros_like(cnt_acc)

    x = x_ref[...]
    w = w_ref[...]
    logits = jax.lax.dot_general(
        w, x, (((1,), (1,)), ((), ())), preferred_element_type=jnp.float32
    )

    ids = jax.lax.broadcasted_iota(jnp.int32, logits.shape, 0).astype(jnp.float32)
    m1 = jnp.max(logits, axis=0, keepdims=True)
    a1 = jnp.min(jnp.where(logits == m1, ids, 8.0), axis=0,
                 keepdims=True)
    masked = jnp.where(ids == a1, -jnp.inf, logits)
    m2 = jnp.max(masked, axis=0, keepdims=True)
    a2 = jnp.min(jnp.where(masked == m2, ids, 8.0), axis=0,
                 keepdims=True)

    g = jnp.exp(m2 - m1)
    rden = 1.0 / (1.0 + g)
    w1 = rden
    w2 = g * rden
    out_ref[...] = jnp.concatenate([w1, w2, a1, a2], axis=0)

    p = jnp.exp(logits - m1)
    p = p * (1.0 / jnp.sum(p, axis=0, keepdims=True))
    psum_acc[...] += jnp.sum(p, axis=1, keepdims=True)
    onehot = (ids == a1).astype(jnp.float32) + (ids == a2).astype(jnp.float32)
    cnt_acc[...] += jnp.sum(onehot, axis=1, keepdims=True)

    @pl.when(i == nsteps - 1)
    def _finish():
        f = cnt_acc[...] / (_N * _TOP_K)
        pmean = psum_acc[...] / _N
        aux_ref[...] = (_NUM_EXPERTS * jnp.sum(f * pmean)).reshape(1, 1)


@jax.jit
def kernel(hidden_states, W):
    grid = (_N // _BLK,)
    packed, aux = pl.pallas_call(
        _router_block,
        grid=grid,
        in_specs=[
            pl.BlockSpec((_BLK, _EMBED), lambda i: (i, 0)),
            pl.BlockSpec((_NUM_EXPERTS, _EMBED), lambda i: (0, 0)),
        ],
        out_specs=[
            pl.BlockSpec((4, _BLK), lambda i: (0, i)),
            pl.BlockSpec((1, 1), lambda i: (0, 0)),
        ],
        out_shape=[
            jax.ShapeDtypeStruct((4, _N), jnp.float32),
            jax.ShapeDtypeStruct((1, 1), jnp.float32),
        ],
        scratch_shapes=[
            pltpu.VMEM((_NUM_EXPERTS, 1), jnp.float32),
            pltpu.VMEM((_NUM_EXPERTS, 1), jnp.float32),
        ],
    )(hidden_states, W)
    wts = packed[0:2].T
    exps = packed[2:4].T.astype(jnp.int32)
    return wts, exps, aux[0, 0]
